# baseline (speedup 1.0000x reference)
import functools
import jax
import jax.numpy as jnp
from jax import lax
from jax.experimental import pallas as pl
from jax.experimental.pallas import tpu as pltpu
from jax.experimental.pallas import tpu_sc as plsc

EMBED = 256
HEADS = 8
LEVELS = 4
POINTS = 4
HD = EMBED // HEADS
LP = LEVELS * POINTS
NQ = 5440
BS = 2
ROWS = BS * NQ
BLK = 640
NW = 32
RPW = ROWS // NW
CH = 2


def _tc_pre_body(q_ref, v_ref, rx_ref, ry_ref, boff_ref,
                 wx_ref, bx_ref, wy_ref, by_ref, wa_ref, ba_ref,
                 wv_ref, bv_ref,
                 Wv_ref, Hv_ref, sv_ref, hv_ref, g_ref,
                 idx_ref, w_ref, tab_ref):
    q = q_ref[...]
    tab_ref[...] = jnp.dot(v_ref[...], wv_ref[...],
                           preferred_element_type=jnp.float32) + bv_ref[...]
    sox = jnp.dot(q, wx_ref[...], preferred_element_type=jnp.float32) + bx_ref[...]
    soy = jnp.dot(q, wy_ref[...], preferred_element_type=jnp.float32) + by_ref[...]
    logits = jnp.dot(q, wa_ref[...], preferred_element_type=jnp.float32) + ba_ref[...]
    m = jnp.max(logits, axis=1, keepdims=True)
    e = jnp.exp(logits - m)
    s = lax.dot_general(e, g_ref[...], (((1,), (0,)), ((), ())),
                        precision=lax.Precision.HIGHEST)
    aw = e / s
    Wv = Wv_ref[...]
    Hv = Hv_ref[...]
    gx = 2.0 * (rx_ref[...] + sox / Wv) - 1.0
    gy = 2.0 * (ry_ref[...] + soy / Hv) - 1.0
    x = ((gx + 1.0) * Wv - 1.0) * 0.5
    y = ((gy + 1.0) * Hv - 1.0) * 0.5
    x0f = jnp.floor(x)
    y0f = jnp.floor(y)
    fx = x - x0f
    fy = y - y0f
    x0 = x0f.astype(jnp.int32)
    y0 = y0f.astype(jnp.int32)
    Wi = Wv.astype(jnp.int32)
    Hi = Hv.astype(jnp.int32)
    sv = sv_ref[...]
    hv = hv_ref[...]
    boff = boff_ref[...]
    for c, (cy, cx) in enumerate(((0, 0), (0, 1), (1, 0), (1, 1))):
        xc = x0 + cx
        yc = y0 + cy
        wgt = aw * (fx if cx else 1.0 - fx) * (fy if cy else 1.0 - fy)
        valid = (xc >= 0) & (xc < Wi) & (yc >= 0) & (yc < Hi)
        wgt = jnp.where(valid, wgt, 0.0)
        pos = jnp.where(valid, sv + yc * Wi + xc, 0)
        idx_ref[:, pl.ds(c * 128, 128)] = boff + pos * HEADS + hv
        w_ref[:, pl.ds(c * 128, 128)] = wgt


def _tc_out_body(x_ref, w_ref, b_ref, o_ref):
    o_ref[...] = jnp.dot(x_ref[...], w_ref[...],
                         preferred_element_type=jnp.float32) + b_ref[...]


def _sc_body(tab_hbm, tabv_hbm, idx_hbm, w_hbm, out_hbm, idx_v, w_v, rows_v,
             out_v, sem_i0, sem_i1, sem_g0, sem_g1, sem_o0, sem_o1):
    wid = lax.axis_index("s") * 2 + lax.axis_index("c")
    base = wid * RPW
    npairs = RPW // (2 * CH)
    sem_i = (sem_i0, sem_i1)
    sem_g = (sem_g0, sem_g1)
    sem_o = (sem_o0, sem_o1)

    def start_iw(g, s):
        r0 = base + g * CH
        pltpu.async_copy(idx_hbm.at[pl.ds(r0, CH)], idx_v.at[s], sem_i[s])
        pltpu.async_copy(w_hbm.at[pl.ds(r0, CH)], w_v.at[s], sem_i[s])

    def wait_iw(s):
        pltpu.make_async_copy(idx_hbm.at[pl.ds(base, CH)], idx_v.at[s],
                              sem_i[s]).wait()
        pltpu.make_async_copy(w_hbm.at[pl.ds(base, CH)], w_v.at[s],
                              sem_i[s]).wait()

    def fire_gathers(s):
        for r in range(CH):
            for c in range(4):
                pltpu.async_copy(
                    tab_hbm.at[idx_v.at[s, r, pl.ds(c * 128, 128)]],
                    rows_v.at[s, r, pl.ds(c * 128, 128)], sem_g[s])

    def wait_gathers(s):
        pltpu.make_async_copy(tabv_hbm.at[0], rows_v.at[s], sem_g[s]).wait()

    def compute(g, s):
        for r in range(CH):
            def hbody(h, carry2):
                z = jnp.zeros((16,), jnp.float32)
                acc0 = [z, z, z, z]
                acc1 = [z, z, z, z]
                j0 = h * LP
                for c in range(4):
                    wv16 = w_v[s, r, pl.ds(c * 128 + j0, LP)]
                    for k in range(LP):
                        wsc = wv16[k]
                        j = c * 128 + j0 + k
                        acc0[c] = acc0[c] + wsc * rows_v[s, r, j, pl.ds(0, 16)]
                        acc1[c] = acc1[c] + wsc * rows_v[s, r, j, pl.ds(16, 16)]
                out_v[s, r, pl.ds(h * HD, 16)] = \
                    (acc0[0] + acc0[1]) + (acc0[2] + acc0[3])
                out_v[s, r, pl.ds(h * HD + 16, 16)] = \
                    (acc1[0] + acc1[1]) + (acc1[2] + acc1[3])
                return carry2
            lax.fori_loop(0, HEADS, hbody, 0)
        pltpu.async_copy(out_v.at[s], out_hbm.at[pl.ds(base + g * CH, CH)],
                         sem_o[s])

    def wait_out(s):
        pltpu.make_async_copy(out_v.at[s], out_hbm.at[pl.ds(base, CH)],
                              sem_o[s]).wait()

    nch = 2 * npairs
    pltpu.sync_copy(idx_hbm.at[pl.ds(base, CH)], idx_v.at[0])
    pltpu.sync_copy(w_hbm.at[pl.ds(base, CH)], w_v.at[0])
    fire_gathers(0)
    start_iw(1, 1)
    pltpu.async_copy(out_hbm.at[pl.ds(base, CH)], out_v.at[0], sem_o[0])
    pltpu.async_copy(out_hbm.at[pl.ds(base, CH)], out_v.at[1], sem_o[1])

    def pair(p, carry):
        g0 = 2 * p
        wait_iw(1)
        wait_gathers(0)
        fire_gathers(1)
        wait_out(0)
        compute(g0, 0)
        start_iw(jnp.minimum(g0 + 2, nch - 1), 0)
        wait_gathers(1)
        wait_iw(0)
        fire_gathers(0)
        wait_out(1)
        compute(g0 + 1, 1)
        start_iw(jnp.minimum(g0 + 3, nch - 1), 1)
        return carry

    lax.fori_loop(0, npairs, pair, 0)
    wait_iw(1)
    wait_gathers(0)
    wait_out(0)
    wait_out(1)


def kernel(query, value, reference_points, spatial_shapes, level_start_index,
           W_samp, b_samp, W_attn, b_attn, W_val, b_val, W_out, b_out):
    q = query.reshape(ROWS, EMBED)
    v = value.reshape(ROWS, EMBED)

    lane = jnp.arange(128, dtype=jnp.int32)
    lvl = (lane // POINTS) % LEVELS
    ssf = spatial_shapes.astype(jnp.float32)
    Wv = ssf[:, 1][lvl][None, :]
    Hv = ssf[:, 0][lvl][None, :]
    sv = level_start_index[lvl][None, :].astype(jnp.int32)
    hv = (lane // LP)[None, :]
    G = (jnp.arange(128)[:, None] // LP ==
         jnp.arange(128)[None, :] // LP).astype(jnp.float32)
    rx_b = reference_points[..., 0][:, :, lvl].reshape(ROWS, 128)
    ry_b = reference_points[..., 1][:, :, lvl].reshape(ROWS, 128)
    boff = (jnp.arange(ROWS, dtype=jnp.int32)[:, None] // NQ) * (NQ * HEADS)
    W_x = W_samp[:, 0::2]
    W_y = W_samp[:, 1::2]
    b_x = b_samp[0::2][None, :]
    b_y = b_samp[1::2][None, :]
    ba = b_attn[None, :]
    bv = b_val[None, :]
    bo = b_out[None, :]

    nblk = ROWS // BLK
    row_spec = lambda c: pl.BlockSpec((BLK, c), lambda i: (i, 0))
    full_spec = lambda r, c: pl.BlockSpec((r, c), lambda i: (0, 0))

    idx, w, tab = pl.pallas_call(
        _tc_pre_body,
        grid=(nblk,),
        in_specs=[
            row_spec(EMBED),
            row_spec(EMBED),
            row_spec(128),
            row_spec(128),
            row_spec(1),
            full_spec(EMBED, 128),
            full_spec(1, 128),
            full_spec(EMBED, 128),
            full_spec(1, 128),
            full_spec(EMBED, 128),
            full_spec(1, 128),
            full_spec(EMBED, EMBED),
            full_spec(1, EMBED),
            full_spec(1, 128),
            full_spec(1, 128),
            full_spec(1, 128),
            full_spec(1, 128),
            full_spec(128, 128),
        ],
        out_specs=[
            pl.BlockSpec((BLK, 512), lambda i: (i, 0)),
            pl.BlockSpec((BLK, 512), lambda i: (i, 0)),
            pl.BlockSpec((BLK, EMBED), lambda i: (i, 0)),
        ],
        out_shape=[
            jax.ShapeDtypeStruct((ROWS, 512), jnp.int32),
            jax.ShapeDtypeStruct((ROWS, 512), jnp.float32),
            jax.ShapeDtypeStruct((ROWS, EMBED), jnp.float32),
        ],
    )(q, v, rx_b, ry_b, boff, W_x, b_x, W_y, b_y, W_attn, ba, W_val, bv,
      Wv, Hv, sv, hv, G)

    table = tab.reshape(ROWS * HEADS, HD)

    sc_call = functools.partial(
        pl.kernel,
        out_type=jax.ShapeDtypeStruct((ROWS, EMBED), jnp.float32),
        mesh=plsc.VectorSubcoreMesh(core_axis_name="c", subcore_axis_name="s"),
        scratch_types=[
            pltpu.VMEM((2, CH, 512), jnp.int32),
            pltpu.VMEM((2, CH, 512), jnp.float32),
            pltpu.VMEM((2, CH, 512, HD), jnp.float32),
            pltpu.VMEM((2, CH, EMBED), jnp.float32),
            pltpu.SemaphoreType.DMA,
            pltpu.SemaphoreType.DMA,
            pltpu.SemaphoreType.DMA,
            pltpu.SemaphoreType.DMA,
            pltpu.SemaphoreType.DMA,
            pltpu.SemaphoreType.DMA,
        ],
        compiler_params=pltpu.CompilerParams(use_tc_tiling_on_sc=False),
    )(_sc_body)
    tabv = jnp.zeros((1, CH, 512, HD), jnp.float32)
    msda = sc_call(table, tabv, idx, w)

    out = pl.pallas_call(
        _tc_out_body,
        grid=(nblk,),
        in_specs=[
            row_spec(EMBED),
            full_spec(EMBED, EMBED),
            full_spec(1, EMBED),
        ],
        out_specs=pl.BlockSpec((BLK, EMBED), lambda i: (i, 0)),
        out_shape=jax.ShapeDtypeStruct((ROWS, EMBED), jnp.float32),
    )(msda, W_out, bo)

    return out.reshape(BS, NQ, EMBED)


# speedup vs baseline: 1.3143x; 1.3143x over previous
import functools
import jax
import jax.numpy as jnp
from jax import lax
from jax.experimental import pallas as pl
from jax.experimental.pallas import tpu as pltpu
from jax.experimental.pallas import tpu_sc as plsc

EMBED = 256
HEADS = 8
LEVELS = 4
POINTS = 4
HD = EMBED // HEADS
LP = LEVELS * POINTS
NQ = 5440
BS = 2
ROWS = BS * NQ
BLK = 640
NW = 32
SPATIAL = ((64, 64), (32, 32), (16, 16), (8, 8))
LEVEL_START = (0, 4096, 5120, 5376)
RPW = ROWS // NW
CH = 2


def _tc_pre_body(q_ref, v_ref, rx_ref, ry_ref, boff_ref,
                 wx_ref, bx_ref, wy_ref, by_ref, wa_ref, ba_ref,
                 wv_ref, bv_ref,
                 Wv_ref, Hv_ref, sv_ref, hvo_ref, g_ref,
                 idx_ref, w_ref, tab_ref):
    q = q_ref[...]
    tab_ref[...] = jnp.dot(v_ref[...], wv_ref[...],
                           preferred_element_type=jnp.float32) + bv_ref[...]
    sox = jnp.dot(q, wx_ref[...], preferred_element_type=jnp.float32) + bx_ref[...]
    soy = jnp.dot(q, wy_ref[...], preferred_element_type=jnp.float32) + by_ref[...]
    logits = jnp.dot(q, wa_ref[...], preferred_element_type=jnp.float32) + ba_ref[...]
    m = jnp.max(logits, axis=1, keepdims=True)
    e = jnp.exp(logits - m)
    s = lax.dot_general(e, g_ref[...], (((1,), (0,)), ((), ())),
                        precision=lax.Precision.HIGHEST)
    aw = e / s
    Wv = Wv_ref[...]
    Hv = Hv_ref[...]
    gx = 2.0 * (rx_ref[...] + sox / Wv) - 1.0
    gy = 2.0 * (ry_ref[...] + soy / Hv) - 1.0
    x = ((gx + 1.0) * Wv - 1.0) * 0.5
    y = ((gy + 1.0) * Hv - 1.0) * 0.5
    x0f = jnp.floor(x)
    y0f = jnp.floor(y)
    fx = x - x0f
    fy = y - y0f
    x0 = x0f.astype(jnp.int32)
    y0 = y0f.astype(jnp.int32)
    Wi = Wv.astype(jnp.int32)
    Hi = Hv.astype(jnp.int32)
    sv = sv_ref[...]
    hvo = hvo_ref[...]
    boff = boff_ref[...]
    wx0 = jnp.where((x0 >= 0) & (x0 < Wi), 1.0 - fx, 0.0)
    wx1 = jnp.where((x0 + 1 >= 0) & (x0 + 1 < Wi), fx, 0.0)
    wy0 = jnp.where((y0 >= 0) & (y0 < Hi), 1.0 - fy, 0.0)
    wy1 = jnp.where((y0 + 1 >= 0) & (y0 + 1 < Hi), fy, 0.0)
    sx = x0 < 0
    sy = y0 < 0
    wxs0 = jnp.where(sx, wx1, wx0)
    wxs1 = jnp.where(sx, 0.0, wx1)
    wys0 = jnp.where(sy, wy1, wy0)
    wys1 = jnp.where(sy, 0.0, wy1)
    xb = jnp.clip(x0, 0, Wi - 1)
    yb = jnp.clip(y0, 0, Hi - 1)
    idx_ref[...] = boff + hvo + sv + yb * Wi + xb
    for c, (wy_, wx_) in enumerate(((wys0, wxs0), (wys0, wxs1),
                                    (wys1, wxs0), (wys1, wxs1))):
        w_ref[:, pl.ds(c * 128, 128)] = aw * wy_ * wx_


def _tc_out_body(x_ref, w_ref, b_ref, o_ref):
    o_ref[...] = jnp.dot(x_ref[...], w_ref[...],
                         preferred_element_type=jnp.float32) + b_ref[...]


def _sc_body(tab_hbm, tabv_hbm, idx_hbm, w_hbm, out_hbm, idx_v, w_v, rows_v,
             out_v, sem_i0, sem_i1, sem_g0, sem_g1, sem_o0, sem_o1):
    wid = lax.axis_index("s") * 2 + lax.axis_index("c")
    base = wid * RPW
    npairs = RPW // (2 * CH)
    sem_i = (sem_i0, sem_i1)
    sem_g = (sem_g0, sem_g1)
    sem_o = (sem_o0, sem_o1)

    def start_iw(g, s):
        r0 = base + g * CH
        pltpu.async_copy(idx_hbm.at[pl.ds(r0, CH)], idx_v.at[s], sem_i[s])
        pltpu.async_copy(w_hbm.at[pl.ds(r0, CH)], w_v.at[s], sem_i[s])

    def wait_iw(s):
        pltpu.make_async_copy(idx_hbm.at[pl.ds(base, CH)], idx_v.at[s],
                              sem_i[s]).wait()
        pltpu.make_async_copy(w_hbm.at[pl.ds(base, CH)], w_v.at[s],
                              sem_i[s]).wait()

    def fire_gathers(s):
        for r in range(CH):
            pltpu.async_copy(tab_hbm.at[idx_v.at[s, r]], rows_v.at[s, r],
                             sem_g[s])

    def wait_gathers(s):
        pltpu.make_async_copy(tabv_hbm.at[0], rows_v.at[s], sem_g[s]).wait()

    def compute(g, s):
        for r in range(CH):
            def hbody(h, carry2):
                z = jnp.zeros((16,), jnp.float32)
                acc0 = [z, z, z, z]
                acc1 = [z, z, z, z]
                j0 = h * LP
                wv = [w_v[s, r, pl.ds(c * 128 + j0, LP)] for c in range(4)]
                for k in range(LP):
                    for c in range(4):
                        wsc = wv[c][k]
                        acc0[c] = acc0[c] + wsc * \
                            rows_v[s, r, j0 + k, pl.ds(c * HD, 16)]
                        acc1[c] = acc1[c] + wsc * \
                            rows_v[s, r, j0 + k, pl.ds(c * HD + 16, 16)]
                out_v[s, r, pl.ds(h * HD, 16)] = \
                    (acc0[0] + acc0[1]) + (acc0[2] + acc0[3])
                out_v[s, r, pl.ds(h * HD + 16, 16)] = \
                    (acc1[0] + acc1[1]) + (acc1[2] + acc1[3])
                return carry2
            lax.fori_loop(0, HEADS, hbody, 0)
        pltpu.async_copy(out_v.at[s], out_hbm.at[pl.ds(base + g * CH, CH)],
                         sem_o[s])

    def wait_out(s):
        pltpu.make_async_copy(out_v.at[s], out_hbm.at[pl.ds(base, CH)],
                              sem_o[s]).wait()

    nch = 2 * npairs
    pltpu.sync_copy(idx_hbm.at[pl.ds(base, CH)], idx_v.at[0])
    pltpu.sync_copy(w_hbm.at[pl.ds(base, CH)], w_v.at[0])
    fire_gathers(0)
    start_iw(1, 1)
    pltpu.async_copy(out_hbm.at[pl.ds(base, CH)], out_v.at[0], sem_o[0])
    pltpu.async_copy(out_hbm.at[pl.ds(base, CH)], out_v.at[1], sem_o[1])

    def pair(p, carry):
        g0 = 2 * p
        wait_iw(1)
        wait_gathers(0)
        fire_gathers(1)
        wait_out(0)
        compute(g0, 0)
        start_iw(jnp.minimum(g0 + 2, nch - 1), 0)
        wait_gathers(1)
        wait_iw(0)
        fire_gathers(0)
        wait_out(1)
        compute(g0 + 1, 1)
        start_iw(jnp.minimum(g0 + 3, nch - 1), 1)
        return carry

    lax.fori_loop(0, npairs, pair, 0)
    wait_iw(1)
    wait_gathers(0)
    wait_out(0)
    wait_out(1)


def kernel(query, value, reference_points, spatial_shapes, level_start_index,
           W_samp, b_samp, W_attn, b_attn, W_val, b_val, W_out, b_out):
    q = query.reshape(ROWS, EMBED)
    v = value.reshape(ROWS, EMBED)

    lane = jnp.arange(128, dtype=jnp.int32)
    lvl = (lane // POINTS) % LEVELS
    ssf = spatial_shapes.astype(jnp.float32)
    Wv = ssf[:, 1][lvl][None, :]
    Hv = ssf[:, 0][lvl][None, :]
    sv = level_start_index[lvl][None, :].astype(jnp.int32)
    hvo = ((lane // LP) * NQ)[None, :]
    G = (jnp.arange(128)[:, None] // LP ==
         jnp.arange(128)[None, :] // LP).astype(jnp.float32)
    rx_b = reference_points[..., 0][:, :, lvl].reshape(ROWS, 128)
    ry_b = reference_points[..., 1][:, :, lvl].reshape(ROWS, 128)
    boff = (jnp.arange(ROWS, dtype=jnp.int32)[:, None] // NQ) * (NQ * HEADS)
    W_x = W_samp[:, 0::2]
    W_y = W_samp[:, 1::2]
    b_x = b_samp[0::2][None, :]
    b_y = b_samp[1::2][None, :]
    ba = b_attn[None, :]
    bv = b_val[None, :]
    bo = b_out[None, :]

    nblk = ROWS // BLK
    row_spec = lambda c: pl.BlockSpec((BLK, c), lambda i: (i, 0))
    full_spec = lambda r, c: pl.BlockSpec((r, c), lambda i: (0, 0))

    idx, w, tab = pl.pallas_call(
        _tc_pre_body,
        grid=(nblk,),
        in_specs=[
            row_spec(EMBED),
            row_spec(EMBED),
            row_spec(128),
            row_spec(128),
            row_spec(1),
            full_spec(EMBED, 128),
            full_spec(1, 128),
            full_spec(EMBED, 128),
            full_spec(1, 128),
            full_spec(EMBED, 128),
            full_spec(1, 128),
            full_spec(EMBED, EMBED),
            full_spec(1, EMBED),
            full_spec(1, 128),
            full_spec(1, 128),
            full_spec(1, 128),
            full_spec(1, 128),
            full_spec(128, 128),
        ],
        out_specs=[
            pl.BlockSpec((BLK, 128), lambda i: (i, 0)),
            pl.BlockSpec((BLK, 512), lambda i: (i, 0)),
            pl.BlockSpec((BLK, EMBED), lambda i: (i, 0)),
        ],
        out_shape=[
            jax.ShapeDtypeStruct((ROWS, 128), jnp.int32),
            jax.ShapeDtypeStruct((ROWS, 512), jnp.float32),
            jax.ShapeDtypeStruct((ROWS, EMBED), jnp.float32),
        ],
    )(q, v, rx_b, ry_b, boff, W_x, b_x, W_y, b_y, W_attn, ba, W_val, bv,
      Wv, Hv, sv, hvo, G)

    t = tab.reshape(BS, NQ, HEADS, HD).transpose(0, 2, 1, 3)
    pieces = []
    for (H_, W_), s0 in zip(SPATIAL, LEVEL_START):
        tl = lax.slice_in_dim(t, s0, s0 + H_ * W_, axis=2)
        tl = tl.reshape(BS, HEADS, H_, W_, HD)
        tp = jnp.pad(tl, ((0, 0), (0, 0), (0, 1), (0, 1), (0, 0)))
        patch = jnp.concatenate(
            [tp[:, :, dy:dy + H_, dx:dx + W_]
             for dy in (0, 1) for dx in (0, 1)], axis=-1)
        pieces.append(patch.reshape(BS, HEADS, H_ * W_, 4 * HD))
    table = jnp.concatenate(pieces, axis=2).reshape(ROWS * HEADS, 4 * HD)

    sc_call = functools.partial(
        pl.kernel,
        out_type=jax.ShapeDtypeStruct((ROWS, EMBED), jnp.float32),
        mesh=plsc.VectorSubcoreMesh(core_axis_name="c", subcore_axis_name="s"),
        scratch_types=[
            pltpu.VMEM((2, CH, 128), jnp.int32),
            pltpu.VMEM((2, CH, 512), jnp.float32),
            pltpu.VMEM((2, CH, 128, 4 * HD), jnp.float32),
            pltpu.VMEM((2, CH, EMBED), jnp.float32),
            pltpu.SemaphoreType.DMA,
            pltpu.SemaphoreType.DMA,
            pltpu.SemaphoreType.DMA,
            pltpu.SemaphoreType.DMA,
            pltpu.SemaphoreType.DMA,
            pltpu.SemaphoreType.DMA,
        ],
        compiler_params=pltpu.CompilerParams(use_tc_tiling_on_sc=False),
    )(_sc_body)
    tabv = jnp.zeros((1, CH, 128, 4 * HD), jnp.float32)
    msda = sc_call(table, tabv, idx, w)

    out = pl.pallas_call(
        _tc_out_body,
        grid=(nblk,),
        in_specs=[
            row_spec(EMBED),
            full_spec(EMBED, EMBED),
            full_spec(1, EMBED),
        ],
        out_specs=pl.BlockSpec((BLK, EMBED), lambda i: (i, 0)),
        out_shape=jax.ShapeDtypeStruct((ROWS, EMBED), jnp.float32),
    )(msda, W_out, bo)

    return out.reshape(BS, NQ, EMBED)


# speedup vs baseline: 1.9463x; 1.4809x over previous
import functools
import jax
import jax.numpy as jnp
from jax import lax
from jax.experimental import pallas as pl
from jax.experimental.pallas import tpu as pltpu
from jax.experimental.pallas import tpu_sc as plsc

EMBED = 256
HEADS = 8
LEVELS = 4
POINTS = 4
HD = EMBED // HEADS
LP = LEVELS * POINTS
NQ = 5440
BS = 2
ROWS = BS * NQ
BLK = 640
NW = 32
SPATIAL = ((64, 64), (32, 32), (16, 16), (8, 8))
LEVEL_START = (0, 4096, 5120, 5376)
RPW = ROWS // NW
CH = 2


def _tc_pre_body(q_ref, v_ref, rx_ref, ry_ref, boff_ref,
                 wx_ref, bx_ref, wy_ref, by_ref, wa_ref, ba_ref,
                 wv_ref, bv_ref,
                 Wv_ref, Hv_ref, sv_ref, hvo_ref, g_ref,
                 idx_ref, w_ref, tab_ref):
    q = q_ref[...]
    tab_ref[...] = jnp.dot(v_ref[...], wv_ref[...],
                           preferred_element_type=jnp.float32) + bv_ref[...]
    sox = jnp.dot(q, wx_ref[...], preferred_element_type=jnp.float32) + bx_ref[...]
    soy = jnp.dot(q, wy_ref[...], preferred_element_type=jnp.float32) + by_ref[...]
    logits = jnp.dot(q, wa_ref[...], preferred_element_type=jnp.float32) + ba_ref[...]
    m = jnp.max(logits, axis=1, keepdims=True)
    e = jnp.exp(logits - m)
    s = lax.dot_general(e, g_ref[...], (((1,), (0,)), ((), ())),
                        precision=lax.Precision.HIGHEST)
    aw = e / s
    Wv = Wv_ref[...]
    Hv = Hv_ref[...]
    gx = 2.0 * (rx_ref[...] + sox / Wv) - 1.0
    gy = 2.0 * (ry_ref[...] + soy / Hv) - 1.0
    x = ((gx + 1.0) * Wv - 1.0) * 0.5
    y = ((gy + 1.0) * Hv - 1.0) * 0.5
    x0f = jnp.floor(x)
    y0f = jnp.floor(y)
    fx = x - x0f
    fy = y - y0f
    x0 = x0f.astype(jnp.int32)
    y0 = y0f.astype(jnp.int32)
    Wi = Wv.astype(jnp.int32)
    Hi = Hv.astype(jnp.int32)
    sv = sv_ref[...]
    hvo = hvo_ref[...]
    boff = boff_ref[...]
    wx0 = jnp.where((x0 >= 0) & (x0 < Wi), 1.0 - fx, 0.0)
    wx1 = jnp.where((x0 + 1 >= 0) & (x0 + 1 < Wi), fx, 0.0)
    wy0 = jnp.where((y0 >= 0) & (y0 < Hi), 1.0 - fy, 0.0)
    wy1 = jnp.where((y0 + 1 >= 0) & (y0 + 1 < Hi), fy, 0.0)
    sx = x0 < 0
    sy = y0 < 0
    wxs0 = jnp.where(sx, wx1, wx0)
    wxs1 = jnp.where(sx, 0.0, wx1)
    wys0 = jnp.where(sy, wy1, wy0)
    wys1 = jnp.where(sy, 0.0, wy1)
    xb = jnp.clip(x0, 0, Wi - 1)
    yb = jnp.clip(y0, 0, Hi - 1)
    idx_ref[...] = boff + hvo + sv + yb * Wi + xb
    for c, (wy_, wx_) in enumerate(((wys0, wxs0), (wys0, wxs1),
                                    (wys1, wxs0), (wys1, wxs1))):
        w_ref[:, pl.ds(c * 128, 128)] = aw * wy_ * wx_


def _tc_patch_body(tab_ref, out_ref):
    slab = tab_ref[0, 0]
    out_ref[:, pl.ds(0, HD)] = slab[0:NQ, :]
    for (H_, W_), s0 in zip(SPATIAL, LEVEL_START):
        hw = H_ * W_
        for ci, sh in enumerate((1, W_, W_ + 1)):
            out_ref[pl.ds(s0, hw), pl.ds(HD * (ci + 1), HD)] = \
                slab[s0 + sh:s0 + sh + hw, :]


def _tc_out_body(x_ref, w_ref, b_ref, o_ref):
    o_ref[...] = jnp.dot(x_ref[...], w_ref[...],
                         preferred_element_type=jnp.float32) + b_ref[...]


def _sc_body(tab_hbm, tabv_hbm, idx_hbm, w_hbm, out_hbm, idx_v, w_v, rows_v,
             out_v, sem_i0, sem_i1, sem_g0, sem_g1, sem_o0, sem_o1):
    wid = lax.axis_index("s") * 2 + lax.axis_index("c")
    base = wid * RPW
    npairs = RPW // (2 * CH)
    sem_i = (sem_i0, sem_i1)
    sem_g = (sem_g0, sem_g1)
    sem_o = (sem_o0, sem_o1)

    def start_iw(g, s):
        r0 = base + g * CH
        pltpu.async_copy(idx_hbm.at[pl.ds(r0, CH)], idx_v.at[s], sem_i[s])
        pltpu.async_copy(w_hbm.at[pl.ds(r0, CH)], w_v.at[s], sem_i[s])

    def wait_iw(s):
        pltpu.make_async_copy(idx_hbm.at[pl.ds(base, CH)], idx_v.at[s],
                              sem_i[s]).wait()
        pltpu.make_async_copy(w_hbm.at[pl.ds(base, CH)], w_v.at[s],
                              sem_i[s]).wait()

    def fire_gathers(s):
        for r in range(CH):
            pltpu.async_copy(tab_hbm.at[idx_v.at[s, r]], rows_v.at[s, r],
                             sem_g[s])

    def wait_gathers(s):
        pltpu.make_async_copy(tabv_hbm.at[0], rows_v.at[s], sem_g[s]).wait()

    def compute(g, s):
        for r in range(CH):
            def hbody(h, carry2):
                z = jnp.zeros((16,), jnp.float32)
                acc0 = [z, z, z, z]
                acc1 = [z, z, z, z]
                j0 = h * LP
                wv = [w_v[s, r, pl.ds(c * 128 + j0, LP)] for c in range(4)]
                for k in range(LP):
                    for c in range(4):
                        wsc = wv[c][k]
                        acc0[c] = acc0[c] + wsc * \
                            rows_v[s, r, j0 + k, pl.ds(c * HD, 16)]
                        acc1[c] = acc1[c] + wsc * \
                            rows_v[s, r, j0 + k, pl.ds(c * HD + 16, 16)]
                out_v[s, r, pl.ds(h * HD, 16)] = \
                    (acc0[0] + acc0[1]) + (acc0[2] + acc0[3])
                out_v[s, r, pl.ds(h * HD + 16, 16)] = \
                    (acc1[0] + acc1[1]) + (acc1[2] + acc1[3])
                return carry2
            lax.fori_loop(0, HEADS, hbody, 0)
        pltpu.async_copy(out_v.at[s], out_hbm.at[pl.ds(base + g * CH, CH)],
                         sem_o[s])

    def wait_out(s):
        pltpu.make_async_copy(out_v.at[s], out_hbm.at[pl.ds(base, CH)],
                              sem_o[s]).wait()

    nch = 2 * npairs
    pltpu.sync_copy(idx_hbm.at[pl.ds(base, CH)], idx_v.at[0])
    pltpu.sync_copy(w_hbm.at[pl.ds(base, CH)], w_v.at[0])
    fire_gathers(0)
    start_iw(1, 1)
    pltpu.async_copy(out_hbm.at[pl.ds(base, CH)], out_v.at[0], sem_o[0])
    pltpu.async_copy(out_hbm.at[pl.ds(base, CH)], out_v.at[1], sem_o[1])

    def pair(p, carry):
        g0 = 2 * p
        wait_iw(1)
        wait_gathers(0)
        fire_gathers(1)
        wait_out(0)
        compute(g0, 0)
        start_iw(jnp.minimum(g0 + 2, nch - 1), 0)
        wait_gathers(1)
        wait_iw(0)
        fire_gathers(0)
        wait_out(1)
        compute(g0 + 1, 1)
        start_iw(jnp.minimum(g0 + 3, nch - 1), 1)
        return carry

    lax.fori_loop(0, npairs, pair, 0)
    wait_iw(1)
    wait_gathers(0)
    wait_out(0)
    wait_out(1)


def kernel(query, value, reference_points, spatial_shapes, level_start_index,
           W_samp, b_samp, W_attn, b_attn, W_val, b_val, W_out, b_out):
    q = query.reshape(ROWS, EMBED)
    v = value.reshape(ROWS, EMBED)

    lane = jnp.arange(128, dtype=jnp.int32)
    lvl = (lane // POINTS) % LEVELS
    ssf = spatial_shapes.astype(jnp.float32)
    Wv = ssf[:, 1][lvl][None, :]
    Hv = ssf[:, 0][lvl][None, :]
    sv = level_start_index[lvl][None, :].astype(jnp.int32)
    hvo = ((lane // LP) * NQ)[None, :]
    G = (jnp.arange(128)[:, None] // LP ==
         jnp.arange(128)[None, :] // LP).astype(jnp.float32)
    rx_b = reference_points[..., 0][:, :, lvl].reshape(ROWS, 128)
    ry_b = reference_points[..., 1][:, :, lvl].reshape(ROWS, 128)
    boff = (jnp.arange(ROWS, dtype=jnp.int32)[:, None] // NQ) * (NQ * HEADS)
    W_x = W_samp[:, 0::2]
    W_y = W_samp[:, 1::2]
    b_x = b_samp[0::2][None, :]
    b_y = b_samp[1::2][None, :]
    ba = b_attn[None, :]
    bv = b_val[None, :]
    bo = b_out[None, :]

    nblk = ROWS // BLK
    row_spec = lambda c: pl.BlockSpec((BLK, c), lambda i: (i, 0))
    full_spec = lambda r, c: pl.BlockSpec((r, c), lambda i: (0, 0))

    idx, w, tab = pl.pallas_call(
        _tc_pre_body,
        grid=(nblk,),
        in_specs=[
            row_spec(EMBED),
            row_spec(EMBED),
            row_spec(128),
            row_spec(128),
            row_spec(1),
            full_spec(EMBED, 128),
            full_spec(1, 128),
            full_spec(EMBED, 128),
            full_spec(1, 128),
            full_spec(EMBED, 128),
            full_spec(1, 128),
            full_spec(EMBED, EMBED),
            full_spec(1, EMBED),
            full_spec(1, 128),
            full_spec(1, 128),
            full_spec(1, 128),
            full_spec(1, 128),
            full_spec(128, 128),
        ],
        out_specs=[
            pl.BlockSpec((BLK, 128), lambda i: (i, 0)),
            pl.BlockSpec((BLK, 512), lambda i: (i, 0)),
            pl.BlockSpec((BLK, EMBED), lambda i: (i, 0)),
        ],
        out_shape=[
            jax.ShapeDtypeStruct((ROWS, 128), jnp.int32),
            jax.ShapeDtypeStruct((ROWS, 512), jnp.float32),
            jax.ShapeDtypeStruct((ROWS, EMBED), jnp.float32),
        ],
    )(q, v, rx_b, ry_b, boff, W_x, b_x, W_y, b_y, W_attn, ba, W_val, bv,
      Wv, Hv, sv, hvo, G)

    PAD = 72
    tabp = jnp.pad(
        tab.reshape(BS, NQ, HEADS, HD).transpose(0, 2, 1, 3),
        ((0, 0), (0, 0), (0, PAD), (0, 0)))
    table = pl.pallas_call(
        _tc_patch_body,
        grid=(BS * HEADS,),
        in_specs=[pl.BlockSpec((1, 1, NQ + PAD, HD),
                               lambda p: (p // HEADS, p % HEADS, 0, 0))],
        out_specs=pl.BlockSpec((NQ, 4 * HD), lambda p: (p, 0)),
        out_shape=jax.ShapeDtypeStruct((ROWS * HEADS, 4 * HD), jnp.float32),
    )(tabp)

    sc_call = functools.partial(
        pl.kernel,
        out_type=jax.ShapeDtypeStruct((ROWS, EMBED), jnp.float32),
        mesh=plsc.VectorSubcoreMesh(core_axis_name="c", subcore_axis_name="s"),
        scratch_types=[
            pltpu.VMEM((2, CH, 128), jnp.int32),
            pltpu.VMEM((2, CH, 512), jnp.float32),
            pltpu.VMEM((2, CH, 128, 4 * HD), jnp.float32),
            pltpu.VMEM((2, CH, EMBED), jnp.float32),
            pltpu.SemaphoreType.DMA,
            pltpu.SemaphoreType.DMA,
            pltpu.SemaphoreType.DMA,
            pltpu.SemaphoreType.DMA,
            pltpu.SemaphoreType.DMA,
            pltpu.SemaphoreType.DMA,
        ],
    )(_sc_body)
    tabv = jnp.zeros((1, CH, 128, 4 * HD), jnp.float32)
    msda = sc_call(table, tabv, idx, w)

    out = pl.pallas_call(
        _tc_out_body,
        grid=(nblk,),
        in_specs=[
            row_spec(EMBED),
            full_spec(EMBED, EMBED),
            full_spec(1, EMBED),
        ],
        out_specs=pl.BlockSpec((BLK, EMBED), lambda i: (i, 0)),
        out_shape=jax.ShapeDtypeStruct((ROWS, EMBED), jnp.float32),
    )(msda, W_out, bo)

    return out.reshape(BS, NQ, EMBED)


# speedup vs baseline: 2.3332x; 1.1987x over previous
import functools
import jax
import jax.numpy as jnp
from jax import lax
from jax.experimental import pallas as pl
from jax.experimental.pallas import tpu as pltpu
from jax.experimental.pallas import tpu_sc as plsc

EMBED = 256
HEADS = 8
LEVELS = 4
POINTS = 4
HD = EMBED // HEADS
LP = LEVELS * POINTS
NQ = 5440
BS = 2
ROWS = BS * NQ
BLK = 544
NW = 32
SPATIAL = ((64, 64), (32, 32), (16, 16), (8, 8))
LEVEL_START = (0, 4096, 5120, 5376)
RPW = ROWS // NW
CH = 2


def _tc_pre_body(q_ref, v_ref, rp_ref, sx_ref, sy_ref, boff_ref,
                 wx_ref, bx_ref, wy_ref, by_ref, wa_ref, ba_ref,
                 wv_ref, bv_ref,
                 Wv_ref, Hv_ref, sv_ref, hvo_ref, g_ref,
                 idx_ref, w_ref, tab_ref):
    q = q_ref[...]
    vproj = jnp.dot(v_ref[...], wv_ref[...],
                    preferred_element_type=jnp.float32) + bv_ref[...]
    for h in range(HEADS):
        tab_ref[0, h] = vproj[:, h * HD:(h + 1) * HD]
    rp = rp_ref[...]
    rx = lax.dot_general(rp, sx_ref[...], (((1,), (0,)), ((), ())),
                         precision=lax.Precision.HIGHEST)
    ry = lax.dot_general(rp, sy_ref[...], (((1,), (0,)), ((), ())),
                         precision=lax.Precision.HIGHEST)
    sox = jnp.dot(q, wx_ref[...], preferred_element_type=jnp.float32) + bx_ref[...]
    soy = jnp.dot(q, wy_ref[...], preferred_element_type=jnp.float32) + by_ref[...]
    logits = jnp.dot(q, wa_ref[...], preferred_element_type=jnp.float32) + ba_ref[...]
    m = jnp.max(logits, axis=1, keepdims=True)
    e = jnp.exp(logits - m)
    s = lax.dot_general(e, g_ref[...], (((1,), (0,)), ((), ())),
                        precision=lax.Precision.HIGHEST)
    aw = e / s
    Wv = Wv_ref[...]
    Hv = Hv_ref[...]
    gx = 2.0 * (rx + sox / Wv) - 1.0
    gy = 2.0 * (ry + soy / Hv) - 1.0
    x = ((gx + 1.0) * Wv - 1.0) * 0.5
    y = ((gy + 1.0) * Hv - 1.0) * 0.5
    x0f = jnp.floor(x)
    y0f = jnp.floor(y)
    fx = x - x0f
    fy = y - y0f
    x0 = x0f.astype(jnp.int32)
    y0 = y0f.astype(jnp.int32)
    Wi = Wv.astype(jnp.int32)
    Hi = Hv.astype(jnp.int32)
    sv = sv_ref[...]
    hvo = hvo_ref[...]
    boff = boff_ref[...]
    wx0 = jnp.where((x0 >= 0) & (x0 < Wi), 1.0 - fx, 0.0)
    wx1 = jnp.where((x0 + 1 >= 0) & (x0 + 1 < Wi), fx, 0.0)
    wy0 = jnp.where((y0 >= 0) & (y0 < Hi), 1.0 - fy, 0.0)
    wy1 = jnp.where((y0 + 1 >= 0) & (y0 + 1 < Hi), fy, 0.0)
    sx = x0 < 0
    sy = y0 < 0
    wxs0 = jnp.where(sx, wx1, wx0)
    wxs1 = jnp.where(sx, 0.0, wx1)
    wys0 = jnp.where(sy, wy1, wy0)
    wys1 = jnp.where(sy, 0.0, wy1)
    xb = jnp.clip(x0, 0, Wi - 1)
    yb = jnp.clip(y0, 0, Hi - 1)
    idx_ref[...] = boff + hvo + sv + yb * Wi + xb
    for c, (wy_, wx_) in enumerate(((wys0, wxs0), (wys0, wxs1),
                                    (wys1, wxs0), (wys1, wxs1))):
        w_ref[:, pl.ds(c * 128, 128)] = aw * wy_ * wx_


def _tc_patch_body(tab_ref, out_ref):
    slab = jnp.concatenate(
        [tab_ref[0, 0], jnp.zeros((72, HD), jnp.float32)], axis=0)
    out_ref[:, pl.ds(0, HD)] = slab[0:NQ, :]
    for (H_, W_), s0 in zip(SPATIAL, LEVEL_START):
        hw = H_ * W_
        for ci, sh in enumerate((1, W_, W_ + 1)):
            out_ref[pl.ds(s0, hw), pl.ds(HD * (ci + 1), HD)] = \
                slab[s0 + sh:s0 + sh + hw, :]


def _tc_out_body(x_ref, w_ref, b_ref, o_ref):
    o_ref[...] = jnp.dot(x_ref[...], w_ref[...],
                         preferred_element_type=jnp.float32) + b_ref[...]


def _sc_body(tab_hbm, tabv_hbm, idx_hbm, w_hbm, out_hbm, idx_v, w_v, rows_v,
             out_v, sem_i0, sem_i1, sem_g0, sem_g1, sem_o0, sem_o1):
    wid = lax.axis_index("s") * 2 + lax.axis_index("c")
    base = wid * RPW
    npairs = RPW // (2 * CH)
    sem_i = (sem_i0, sem_i1)
    sem_g = (sem_g0, sem_g1)
    sem_o = (sem_o0, sem_o1)

    def start_iw(g, s):
        r0 = base + g * CH
        pltpu.async_copy(idx_hbm.at[pl.ds(r0, CH)], idx_v.at[s], sem_i[s])
        pltpu.async_copy(w_hbm.at[pl.ds(r0, CH)], w_v.at[s], sem_i[s])

    def wait_iw(s):
        pltpu.make_async_copy(idx_hbm.at[pl.ds(base, CH)], idx_v.at[s],
                              sem_i[s]).wait()
        pltpu.make_async_copy(w_hbm.at[pl.ds(base, CH)], w_v.at[s],
                              sem_i[s]).wait()

    def fire_gathers(s):
        for r in range(CH):
            pltpu.async_copy(tab_hbm.at[idx_v.at[s, r]], rows_v.at[s, r],
                             sem_g[s])

    def wait_gathers(s):
        pltpu.make_async_copy(tabv_hbm.at[0], rows_v.at[s], sem_g[s]).wait()

    def compute(g, s):
        for r in range(CH):
            def hbody(h, carry2):
                z = jnp.zeros((16,), jnp.float32)
                acc0 = [z, z, z, z]
                acc1 = [z, z, z, z]
                j0 = h * LP
                wv = [w_v[s, r, pl.ds(c * 128 + j0, LP)] for c in range(4)]
                for k in range(LP):
                    for c in range(4):
                        wsc = wv[c][k]
                        acc0[c] = acc0[c] + wsc * \
                            rows_v[s, r, j0 + k, pl.ds(c * HD, 16)]
                        acc1[c] = acc1[c] + wsc * \
                            rows_v[s, r, j0 + k, pl.ds(c * HD + 16, 16)]
                out_v[s, r, pl.ds(h * HD, 16)] = \
                    (acc0[0] + acc0[1]) + (acc0[2] + acc0[3])
                out_v[s, r, pl.ds(h * HD + 16, 16)] = \
                    (acc1[0] + acc1[1]) + (acc1[2] + acc1[3])
                return carry2
            lax.fori_loop(0, HEADS, hbody, 0)
        pltpu.async_copy(out_v.at[s], out_hbm.at[pl.ds(base + g * CH, CH)],
                         sem_o[s])

    def wait_out(s):
        pltpu.make_async_copy(out_v.at[s], out_hbm.at[pl.ds(base, CH)],
                              sem_o[s]).wait()

    nch = 2 * npairs
    pltpu.sync_copy(idx_hbm.at[pl.ds(base, CH)], idx_v.at[0])
    pltpu.sync_copy(w_hbm.at[pl.ds(base, CH)], w_v.at[0])
    fire_gathers(0)
    start_iw(1, 1)
    pltpu.async_copy(out_hbm.at[pl.ds(base, CH)], out_v.at[0], sem_o[0])
    pltpu.async_copy(out_hbm.at[pl.ds(base, CH)], out_v.at[1], sem_o[1])

    def pair(p, carry):
        g0 = 2 * p
        wait_iw(1)
        wait_gathers(0)
        fire_gathers(1)
        wait_out(0)
        compute(g0, 0)
        start_iw(jnp.minimum(g0 + 2, nch - 1), 0)
        wait_gathers(1)
        wait_iw(0)
        fire_gathers(0)
        wait_out(1)
        compute(g0 + 1, 1)
        start_iw(jnp.minimum(g0 + 3, nch - 1), 1)
        return carry

    lax.fori_loop(0, npairs, pair, 0)
    wait_iw(1)
    wait_gathers(0)
    wait_out(0)
    wait_out(1)


def kernel(query, value, reference_points, spatial_shapes, level_start_index,
           W_samp, b_samp, W_attn, b_attn, W_val, b_val, W_out, b_out):
    q = query.reshape(ROWS, EMBED)
    v = value.reshape(ROWS, EMBED)

    lane = jnp.arange(128, dtype=jnp.int32)
    lvl = (lane // POINTS) % LEVELS
    ssf = spatial_shapes.astype(jnp.float32)
    Wv = ssf[:, 1][lvl][None, :]
    Hv = ssf[:, 0][lvl][None, :]
    sv = level_start_index[lvl][None, :].astype(jnp.int32)
    hvo = ((lane // LP) * NQ)[None, :]
    G = (jnp.arange(128)[:, None] // LP ==
         jnp.arange(128)[None, :] // LP).astype(jnp.float32)
    rp8 = reference_points.reshape(ROWS, 2 * LEVELS)
    col = jnp.arange(8)[:, None]
    Sx = (col == 2 * lvl[None, :]).astype(jnp.float32)
    Sy = (col == 2 * lvl[None, :] + 1).astype(jnp.float32)
    boff = (jnp.arange(ROWS, dtype=jnp.int32)[:, None] // NQ) * (NQ * HEADS)
    W_x = W_samp[:, 0::2]
    W_y = W_samp[:, 1::2]
    b_x = b_samp[0::2][None, :]
    b_y = b_samp[1::2][None, :]
    ba = b_attn[None, :]
    bv = b_val[None, :]
    bo = b_out[None, :]

    nblk = ROWS // BLK
    row_spec = lambda c: pl.BlockSpec((BLK, c), lambda i: (i, 0))
    full_spec = lambda r, c: pl.BlockSpec((r, c), lambda i: (0, 0))

    idx, w, tab = pl.pallas_call(
        _tc_pre_body,
        grid=(nblk,),
        in_specs=[
            row_spec(EMBED),
            row_spec(EMBED),
            row_spec(2 * LEVELS),
            full_spec(2 * LEVELS, 128),
            full_spec(2 * LEVELS, 128),
            row_spec(1),
            full_spec(EMBED, 128),
            full_spec(1, 128),
            full_spec(EMBED, 128),
            full_spec(1, 128),
            full_spec(EMBED, 128),
            full_spec(1, 128),
            full_spec(EMBED, EMBED),
            full_spec(1, EMBED),
            full_spec(1, 128),
            full_spec(1, 128),
            full_spec(1, 128),
            full_spec(1, 128),
            full_spec(128, 128),
        ],
        out_specs=[
            pl.BlockSpec((BLK, 128), lambda i: (i, 0)),
            pl.BlockSpec((BLK, 512), lambda i: (i, 0)),
            pl.BlockSpec((1, HEADS, BLK, HD),
                         lambda i: (i // (NQ // BLK), 0, i % (NQ // BLK), 0)),
        ],
        out_shape=[
            jax.ShapeDtypeStruct((ROWS, 128), jnp.int32),
            jax.ShapeDtypeStruct((ROWS, 512), jnp.float32),
            jax.ShapeDtypeStruct((BS, HEADS, NQ, HD), jnp.float32),
        ],
    )(q, v, rp8, Sx, Sy, boff, W_x, b_x, W_y, b_y, W_attn, ba, W_val, bv,
      Wv, Hv, sv, hvo, G)

    table = pl.pallas_call(
        _tc_patch_body,
        grid=(BS * HEADS,),
        in_specs=[pl.BlockSpec((1, 1, NQ, HD),
                               lambda p: (p // HEADS, p % HEADS, 0, 0))],
        out_specs=pl.BlockSpec((NQ, 4 * HD), lambda p: (p, 0)),
        out_shape=jax.ShapeDtypeStruct((ROWS * HEADS, 4 * HD), jnp.float32),
    )(tab)

    sc_call = functools.partial(
        pl.kernel,
        out_type=jax.ShapeDtypeStruct((ROWS, EMBED), jnp.float32),
        mesh=plsc.VectorSubcoreMesh(core_axis_name="c", subcore_axis_name="s"),
        scratch_types=[
            pltpu.VMEM((2, CH, 128), jnp.int32),
            pltpu.VMEM((2, CH, 512), jnp.float32),
            pltpu.VMEM((2, CH, 128, 4 * HD), jnp.float32),
            pltpu.VMEM((2, CH, EMBED), jnp.float32),
            pltpu.SemaphoreType.DMA,
            pltpu.SemaphoreType.DMA,
            pltpu.SemaphoreType.DMA,
            pltpu.SemaphoreType.DMA,
            pltpu.SemaphoreType.DMA,
            pltpu.SemaphoreType.DMA,
        ],
    )(_sc_body)
    tabv = jnp.zeros((1, CH, 128, 4 * HD), jnp.float32)
    msda = sc_call(table, tabv, idx, w)

    out = pl.pallas_call(
        _tc_out_body,
        grid=(nblk,),
        in_specs=[
            row_spec(EMBED),
            full_spec(EMBED, EMBED),
            full_spec(1, EMBED),
        ],
        out_specs=pl.BlockSpec((BLK, EMBED), lambda i: (i, 0)),
        out_shape=jax.ShapeDtypeStruct((ROWS, EMBED), jnp.float32),
    )(msda, W_out, bo)

    return out.reshape(BS, NQ, EMBED)
